# Initial kernel scaffold; baseline (speedup 1.0000x reference)
import jax
import jax.numpy as jnp
from jax.experimental import pallas as pl


def kernel(flow):
    raise NotImplementedError("write your pallas kernel here")


# speedup vs baseline: 1.5809x; 1.5809x over previous
import functools

import jax
import jax.numpy as jnp
from jax import lax
from jax.experimental import pallas as pl
from jax.experimental.pallas import tpu as pltpu
from jax.experimental.pallas import tpu_sc as plsc

N_NODES = 100000
N_EDGES = 6400000

NC = 2
NS = 16
L = 16
NW = NC * NS

E_W = N_EDGES // NW
CHUNK = 2000
WORDS = CHUNK * 3
N_CHUNKS = E_W // CHUNK
GROUPS = CHUNK // L


@functools.partial(
    pl.kernel,
    out_type=jax.ShapeDtypeStruct((NW, N_NODES), jnp.float32),
    mesh=plsc.VectorSubcoreMesh(core_axis_name="c", subcore_axis_name="s"),
    compiler_params=pltpu.CompilerParams(needs_layout_passes=False),
    scratch_types=[
        pltpu.VMEM((WORDS,), jnp.float32),
        pltpu.VMEM((N_NODES,), jnp.float32),
    ],
)
def _scatter_kernel(flow_hbm, out_hbm, buf, acc):
    wid = lax.axis_index("s") * NC + lax.axis_index("c")
    iota3 = lax.iota(jnp.int32, L) * 3

    def zero_body(i, carry):
        acc[pl.ds(i * L, L)] = jnp.zeros((L,), jnp.float32)
        return carry

    lax.fori_loop(0, N_NODES // L, zero_body, 0)

    base_w = wid * (E_W * 3)

    def chunk_body(c, carry):
        pltpu.sync_copy(flow_hbm.at[pl.ds(base_w + c * WORDS, WORDS)], buf)

        def group_body(g, inner):
            pos = iota3 + g * (3 * L)
            s = plsc.load_gather(buf, [pos])
            d = plsc.load_gather(buf, [pos + 1])
            v = plsc.load_gather(buf, [pos + 2])
            plsc.addupdate_scatter(acc, [s.astype(jnp.int32)], v)
            plsc.addupdate_scatter(acc, [d.astype(jnp.int32)], -v)
            return inner

        lax.fori_loop(0, GROUPS, group_body, 0)
        return carry

    lax.fori_loop(0, N_CHUNKS, chunk_body, 0)

    pltpu.sync_copy(acc, out_hbm.at[wid])


def kernel(flow):
    partials = _scatter_kernel(flow.reshape(-1))
    arrs = [partials[i] for i in range(NW)]
    while len(arrs) > 1:
        arrs = [arrs[i] + arrs[i + 1] for i in range(0, len(arrs), 2)]
    net = lax.optimization_barrier(arrs[0])
    return jnp.sum(net)


# speedup vs baseline: 32.9923x; 20.8694x over previous
import functools

import jax
import jax.numpy as jnp
from jax import lax
from jax.experimental import pallas as pl
from jax.experimental.pallas import tpu as pltpu
from jax.experimental.pallas import tpu_sc as plsc

N_NODES = 100000
N_EDGES = 6400000

NC = 2
NS = 16
L = 16
NW = NC * NS

E_W = N_EDGES // NW
CHUNK = 4000
N_CHUNKS = E_W // CHUNK
GROUPS = CHUNK // L


@functools.partial(
    pl.kernel,
    out_type=jax.ShapeDtypeStruct((NW, N_NODES), jnp.float32),
    mesh=plsc.VectorSubcoreMesh(core_axis_name="c", subcore_axis_name="s"),
    compiler_params=pltpu.CompilerParams(needs_layout_passes=False),
    scratch_types=[
        pltpu.VMEM((CHUNK,), jnp.int32),
        pltpu.VMEM((CHUNK,), jnp.int32),
        pltpu.VMEM((CHUNK,), jnp.float32),
        pltpu.VMEM((N_NODES,), jnp.float32),
    ],
)
def _scatter_kernel(src_hbm, dst_hbm, val_hbm, out_hbm, sbuf, dbuf, vbuf, acc):
    wid = lax.axis_index("s") * NC + lax.axis_index("c")

    def zero_body(i, carry):
        acc[pl.ds(i * L, L)] = jnp.zeros((L,), jnp.float32)
        return carry

    lax.fori_loop(0, N_NODES // L, zero_body, 0)

    base = wid * E_W

    def chunk_body(c, carry):
        off = base + c * CHUNK
        pltpu.sync_copy(src_hbm.at[pl.ds(off, CHUNK)], sbuf)
        pltpu.sync_copy(dst_hbm.at[pl.ds(off, CHUNK)], dbuf)
        pltpu.sync_copy(val_hbm.at[pl.ds(off, CHUNK)], vbuf)

        def group_body(g, inner):
            sl = pl.ds(g * L, L)
            s = sbuf[sl]
            d = dbuf[sl]
            v = vbuf[sl]
            plsc.addupdate_scatter(acc, [s], v)
            plsc.addupdate_scatter(acc, [d], -v)
            return inner

        lax.fori_loop(0, GROUPS, group_body, 0)
        return carry

    lax.fori_loop(0, N_CHUNKS, chunk_body, 0)

    pltpu.sync_copy(acc, out_hbm.at[wid])


def kernel(flow):
    src = flow[:, 0].astype(jnp.int32)
    dst = flow[:, 1].astype(jnp.int32)
    val = flow[:, 2]
    partials = _scatter_kernel(src, dst, val)
    arrs = [partials[i] for i in range(NW)]
    while len(arrs) > 1:
        arrs = [arrs[i] + arrs[i + 1] for i in range(0, len(arrs), 2)]
    net = lax.optimization_barrier(arrs[0])
    return jnp.sum(net)


# speedup vs baseline: 45.0985x; 1.3669x over previous
import functools

import jax
import jax.numpy as jnp
from jax import lax
from jax.experimental import pallas as pl
from jax.experimental.pallas import tpu as pltpu
from jax.experimental.pallas import tpu_sc as plsc

N_NODES = 100000
N_EDGES = 6400000

NC = 2
NS = 16
L = 16
NW = NC * NS

E_W = N_EDGES // NW
CHUNK = 4000
N_CHUNKS = E_W // CHUNK
GROUPS = CHUNK // L
UNROLL = 5


@functools.partial(
    pl.kernel,
    out_type=jax.ShapeDtypeStruct((NW, N_NODES), jnp.float32),
    mesh=plsc.VectorSubcoreMesh(core_axis_name="c", subcore_axis_name="s"),
    compiler_params=pltpu.CompilerParams(needs_layout_passes=False),
    scratch_types=[
        pltpu.VMEM((CHUNK,), jnp.int32),
        pltpu.VMEM((CHUNK,), jnp.int32),
        pltpu.VMEM((CHUNK,), jnp.int32),
        pltpu.VMEM((CHUNK,), jnp.int32),
        pltpu.VMEM((CHUNK,), jnp.float32),
        pltpu.VMEM((CHUNK,), jnp.float32),
        pltpu.VMEM((N_NODES,), jnp.float32),
        pltpu.SemaphoreType.DMA,
        pltpu.SemaphoreType.DMA,
    ],
)
def _scatter_kernel(src_hbm, dst_hbm, val_hbm, out_hbm, sbuf0, sbuf1,
                    dbuf0, dbuf1, vbuf0, vbuf1, acc, sem0, sem1):
    wid = lax.axis_index("s") * NC + lax.axis_index("c")
    sems = (sem0, sem1)
    sbufs = (sbuf0, sbuf1)
    dbufs = (dbuf0, dbuf1)
    vbufs = (vbuf0, vbuf1)

    def zero_body(i, carry):
        acc[pl.ds(i * L, L)] = jnp.zeros((L,), jnp.float32)
        return carry

    lax.fori_loop(0, N_NODES // L, zero_body, 0)

    base = wid * E_W

    def start_fetch(c, slot):
        off = base + c * CHUNK
        pltpu.async_copy(src_hbm.at[pl.ds(off, CHUNK)], sbufs[slot], sems[slot])
        pltpu.async_copy(dst_hbm.at[pl.ds(off, CHUNK)], dbufs[slot], sems[slot])
        pltpu.async_copy(val_hbm.at[pl.ds(off, CHUNK)], vbufs[slot], sems[slot])

    def wait_fetch(c, slot):
        off = base + c * CHUNK
        pltpu.make_async_copy(src_hbm.at[pl.ds(off, CHUNK)], sbufs[slot], sems[slot]).wait()
        pltpu.make_async_copy(dst_hbm.at[pl.ds(off, CHUNK)], dbufs[slot], sems[slot]).wait()
        pltpu.make_async_copy(val_hbm.at[pl.ds(off, CHUNK)], vbufs[slot], sems[slot]).wait()

    def process(slot):
        def group_body(i, inner):
            for u in range(UNROLL):
                sl = pl.ds((i * UNROLL + u) * L, L)
                s = sbufs[slot][sl]
                d = dbufs[slot][sl]
                v = vbufs[slot][sl]
                plsc.addupdate_scatter(acc, [s], v)
                plsc.addupdate_scatter(acc, [d], -v)
            return inner

        lax.fori_loop(0, GROUPS // UNROLL, group_body, 0)

    start_fetch(0, 0)

    def ring_body(c2, carry):
        c = c2 * 2
        start_fetch(c + 1, 1)
        wait_fetch(c, 0)
        process(0)

        @pl.when(c + 2 < N_CHUNKS)
        def _():
            start_fetch(c + 2, 0)

        wait_fetch(c + 1, 1)
        process(1)
        return carry

    lax.fori_loop(0, N_CHUNKS // 2, ring_body, 0)

    pltpu.sync_copy(acc, out_hbm.at[wid])


def kernel(flow):
    src = flow[:, 0].astype(jnp.int32)
    dst = flow[:, 1].astype(jnp.int32)
    val = flow[:, 2]
    partials = _scatter_kernel(src, dst, val)
    arrs = [partials[i] for i in range(NW)]
    while len(arrs) > 1:
        arrs = [arrs[i] + arrs[i + 1] for i in range(0, len(arrs), 2)]
    net = lax.optimization_barrier(arrs[0])
    return jnp.sum(net)
